# baseline (speedup 1.0000x reference)
import functools

import jax
import jax.numpy as jnp
from jax import lax
from jax.experimental import pallas as pl
from jax.experimental.pallas import tpu as pltpu
from jax.experimental.pallas import tpu_sc as plsc

NC = 2
NS = 16
CHUNK = 128
NBUF = 6


def _make_sc_scatter(n, d2, n_chunks, rows_pad):
  rows_per_tile = rows_pad // NS
  out_per = (n // NS) // 8 * 8
  tail = n - out_per * NS
  mesh = plsc.VectorSubcoreMesh(core_axis_name="c", subcore_axis_name="s")

  @functools.partial(
      pl.kernel,
      out_type=jax.ShapeDtypeStruct((NC, n, d2), jnp.float32),
      mesh=mesh,
      scratch_types=[
          pltpu.VMEM((n_chunks, CHUNK), jnp.int32),
          pltpu.VMEM((n_chunks, CHUNK), jnp.int32),
          pltpu.VMEM((NBUF, CHUNK, d2), jnp.float32),
          pltpu.VMEM_SHARED((rows_pad, d2), jnp.float32),
          pltpu.SemaphoreType.DMA,
          pltpu.SemaphoreType.DMA,
      ],
      compiler_params=pltpu.CompilerParams(use_tc_tiling_on_sc=False),
  )
  def sc_scatter(msg_hbm, src_hbm, dst_hbm, out_hbm,
                 src_v, dst_v, rows_v, acc, gsem, ssem):
    cid = lax.axis_index("c")
    sid = lax.axis_index("s")
    my_msg = msg_hbm.at[cid]

    idx_cp = (pltpu.async_copy(src_hbm.at[sid], src_v, gsem),
              pltpu.async_copy(dst_hbm.at[sid], dst_v, gsem))

    zeros_v = rows_v.at[0, pl.ds(0, 16)]
    def zrow(i, _):
      def zcol(j, _):
        rows_v[0, i, pl.ds(j * 16, 16)] = jnp.zeros((16,), jnp.float32)
        return 0
      return lax.fori_loop(0, d2 // 16, zcol, 0)
    lax.fori_loop(0, 16, zrow, 0)

    base = sid * rows_per_tile
    def zacc(t, _):
      pltpu.sync_copy(zeros_v, acc.at[pl.ds(base + t * 16, 16)])
      return 0
    lax.fori_loop(0, rows_per_tile // 16, zacc, 0)
    if rows_per_tile % 16:
      pltpu.sync_copy(zeros_v, acc.at[pl.ds(base + rows_per_tile - 16, 16)])
    for cp in idx_cp:
      cp.wait()
    plsc.subcore_barrier()

    for b in range(4):
      pltpu.async_copy(my_msg.at[src_v.at[b]], rows_v.at[b], gsem)

    def wait_gather(b, j):
      pltpu.make_async_copy(my_msg.at[src_v.at[j]], rows_v.at[b], gsem).wait()

    def drain_scatter(b):
      pltpu.make_async_copy(rows_v.at[b], acc.at[dst_v.at[0]], ssem).wait()

    def body(t, _):
      for b in range(NBUF):
        j = t * NBUF + b
        wait_gather(b, j)
        pltpu.async_copy(rows_v.at[b], acc.at[dst_v.at[j]], ssem, add=True)
        @pl.when(jnp.logical_and(j >= 2, j + 4 < n_chunks))
        def _():
          drain_scatter(b)
        @pl.when(j + 4 < n_chunks)
        def _():
          pltpu.async_copy(my_msg.at[src_v.at[j + 4]],
                           rows_v.at[(b + 4) % NBUF], gsem)
      return 0
    lax.fori_loop(0, n_chunks // NBUF, body, 0)
    for b in range(NBUF):
      drain_scatter(b)

    plsc.subcore_barrier()
    obase = sid * out_per
    pltpu.sync_copy(acc.at[pl.ds(obase, out_per)],
                    out_hbm.at[cid, pl.ds(obase, out_per)])
    if tail:
      @pl.when(sid == NS - 1)
      def _():
        pltpu.sync_copy(acc.at[pl.ds(out_per * NS, tail)],
                        out_hbm.at[cid, pl.ds(out_per * NS, tail)])

  return sc_scatter


def _mm(a, w):
  return lax.dot_general(a, w, (((1,), (0,)), ((), ())),
                         preferred_element_type=jnp.float32)


def _split_msg(m, m_ref, d2):
  m_ref[0] = m[:, :d2]
  m_ref[1] = m[:, d2:]


def _tc_init(x, w_in, b_in, w_msg0, b_msg0, block):
  n, d = x.shape
  d2 = d // 2

  def body(x_ref, wi_ref, bi_ref, wm_ref, bm_ref, s_ref, m_ref):
    s = jax.nn.relu(_mm(x_ref[...], wi_ref[...]) + bi_ref[...])
    s_ref[...] = s
    _split_msg(jax.nn.relu(_mm(s, wm_ref[...]) + bm_ref[...]), m_ref, d2)

  grid = n // block
  full = lambda shape: pl.BlockSpec(shape, lambda i: (0, 0))
  return pl.pallas_call(
      body,
      grid=(grid,),
      in_specs=[
          pl.BlockSpec((block, d), lambda i: (i, 0)),
          full((d, d)), full((1, d)), full((d, d)), full((1, d)),
      ],
      out_specs=[pl.BlockSpec((block, d), lambda i: (i, 0)),
                 pl.BlockSpec((NC, block, d2), lambda i: (0, i, 0))],
      out_shape=[jax.ShapeDtypeStruct((n, d), jnp.float32),
                 jax.ShapeDtypeStruct((NC, n, d2), jnp.float32)],
  )(x, w_in, b_in.reshape(1, d), w_msg0, b_msg0.reshape(1, d))


def _tc_gru(state, agg, w_sa, w_aa, b_ru, wt_s, wt_b, b_t, wm, bm, block):
  n, d = state.shape
  d2 = d // 2
  emit_msg = wm is not None

  def body(s_ref, a_ref, wsa_ref, waa_ref, bru_ref, wts_ref, wtb_ref,
           bt_ref, *rest):
    if emit_msg:
      wm_ref, bm_ref, ns_ref, m_ref = rest
    else:
      (ns_ref,) = rest
    s = s_ref[...]
    a = jnp.concatenate([a_ref[0], a_ref[1]], axis=1)
    ru = jax.nn.sigmoid(_mm(s, wsa_ref[...]) + _mm(a, waa_ref[...])
                        + bru_ref[...])
    r = ru[:, :d]
    u = ru[:, d:]
    c = jnp.tanh(_mm(s * r, wts_ref[...]) + _mm(a, wtb_ref[...]) + bt_ref[...])
    ns = s * (1.0 - u) + c * u
    ns_ref[...] = ns
    if emit_msg:
      _split_msg(jax.nn.relu(_mm(ns, wm_ref[...]) + bm_ref[...]), m_ref, d2)

  grid = n // block
  full = lambda shape: pl.BlockSpec(shape, lambda i: (0, 0))
  in_specs = [
      pl.BlockSpec((block, d), lambda i: (i, 0)),
      pl.BlockSpec((NC, block, d2), lambda i: (0, i, 0)),
      full((d, 2 * d)), full((d, 2 * d)), full((1, 2 * d)),
      full((d, d)), full((d, d)), full((1, d)),
  ]
  args = [state, agg, w_sa, w_aa, b_ru.reshape(1, 2 * d),
          wt_s, wt_b, b_t.reshape(1, d)]
  if emit_msg:
    in_specs += [full((d, d)), full((1, d))]
    args += [wm, bm.reshape(1, d)]
    out_specs = [pl.BlockSpec((block, d), lambda i: (i, 0)),
                 pl.BlockSpec((NC, block, d2), lambda i: (0, i, 0))]
    out_shape = [jax.ShapeDtypeStruct((n, d), jnp.float32),
                 jax.ShapeDtypeStruct((NC, n, d2), jnp.float32)]
  else:
    out_specs = [pl.BlockSpec((block, d), lambda i: (i, 0))]
    out_shape = [jax.ShapeDtypeStruct((n, d), jnp.float32)]
  return pl.pallas_call(
      body, grid=(grid,), in_specs=in_specs,
      out_specs=out_specs, out_shape=out_shape,
  )(*args)


def kernel(x, edge_index, batch, w_in, b_in, w_msg, b_msg,
           w_r, b_r, w_u, b_u, w_t, b_t):
  n, d = x.shape
  d2 = d // 2
  e = edge_index.shape[1]
  rounds = w_msg.shape[0]
  block = 1000 if n % 1000 == 0 else n // 8

  n_chunks = -(-e // (NS * CHUNK * NBUF)) * NBUF
  pe = NS * n_chunks * CHUNK
  pad = pe - e
  src = edge_index[0]
  dst = edge_index[1]
  if pad:
    fill = jnp.arange(pad, dtype=jnp.int32) % 16
    src = jnp.concatenate([src, fill])
    dst = jnp.concatenate([dst, n + fill])
  src_p = src.reshape(NS, n_chunks, CHUNK)
  dst_p = dst.reshape(NS, n_chunks, CHUNK)
  rows_pad = n + 16
  sc_scatter = _make_sc_scatter(n, d2, n_chunks, rows_pad)

  w_sa = jnp.concatenate([w_r[:, :d, :], w_u[:, :d, :]], axis=2)
  w_aa = jnp.concatenate([w_r[:, d:, :], w_u[:, d:, :]], axis=2)
  b_ru = jnp.concatenate([b_r, b_u], axis=1)
  wt_s = w_t[:, :d, :]
  wt_b = w_t[:, d:, :]

  state, msg = _tc_init(x, w_in, b_in, w_msg[0], b_msg[0], block)
  for r in range(rounds):
    agg = sc_scatter(msg, src_p, dst_p)
    last = r == rounds - 1
    out = _tc_gru(state, agg, w_sa[r], w_aa[r], b_ru[r], wt_s[r], wt_b[r],
                  b_t[r], None if last else w_msg[r + 1],
                  None if last else b_msg[r + 1], block)
    if last:
      state = out[0]
    else:
      state, msg = out
  return state


# speedup vs baseline: 1.0837x; 1.0837x over previous
import functools

import jax
import jax.numpy as jnp
from jax import lax
from jax.experimental import pallas as pl
from jax.experimental.pallas import tpu as pltpu
from jax.experimental.pallas import tpu_sc as plsc

NC = 2
NS = 16
CHUNK = 128
NBUF = 5


def _make_sc_scatter(n, d2, n_chunks, rows_pad):
  rows_per_tile = rows_pad // NS
  out_per = (n // NS) // 8 * 8
  tail = n - out_per * NS
  mesh = plsc.VectorSubcoreMesh(core_axis_name="c", subcore_axis_name="s")

  @functools.partial(
      pl.kernel,
      out_type=jax.ShapeDtypeStruct((NC, n, d2), jnp.float32),
      mesh=mesh,
      scratch_types=[
          pltpu.VMEM((n_chunks, CHUNK), jnp.int32),
          pltpu.VMEM((n_chunks, CHUNK), jnp.int32),
          pltpu.VMEM((NBUF, CHUNK, d2), jnp.float32),
          pltpu.VMEM_SHARED((rows_pad, d2), jnp.float32),
          pltpu.SemaphoreType.DMA,
          pltpu.SemaphoreType.DMA,
      ],
      compiler_params=pltpu.CompilerParams(use_tc_tiling_on_sc=False),
  )
  def sc_scatter(msg_hbm, src_hbm, dst_hbm, out_hbm,
                 src_v, dst_v, rows_v, acc, gsem, ssem):
    cid = lax.axis_index("c")
    sid = lax.axis_index("s")
    my_msg = msg_hbm.at[cid]

    idx_cp = (pltpu.async_copy(src_hbm.at[sid], src_v, gsem),
              pltpu.async_copy(dst_hbm.at[sid], dst_v, gsem))

    zeros_v = rows_v.at[0, pl.ds(0, 16)]
    def zrow(i, _):
      def zcol(j, _):
        rows_v[0, i, pl.ds(j * 16, 16)] = jnp.zeros((16,), jnp.float32)
        return 0
      return lax.fori_loop(0, d2 // 16, zcol, 0)
    lax.fori_loop(0, 16, zrow, 0)

    base = sid * rows_per_tile
    def zacc(t, _):
      pltpu.sync_copy(zeros_v, acc.at[pl.ds(base + t * 16, 16)])
      return 0
    lax.fori_loop(0, rows_per_tile // 16, zacc, 0)
    if rows_per_tile % 16:
      pltpu.sync_copy(zeros_v, acc.at[pl.ds(base + rows_per_tile - 16, 16)])
    for cp in idx_cp:
      cp.wait()
    plsc.subcore_barrier()

    for b in range(4):
      pltpu.async_copy(my_msg.at[src_v.at[b]], rows_v.at[b], gsem)

    def wait_gather(b, j):
      pltpu.make_async_copy(my_msg.at[src_v.at[j]], rows_v.at[b], gsem).wait()

    def drain_scatter(b):
      pltpu.make_async_copy(rows_v.at[b], acc.at[dst_v.at[0]], ssem).wait()

    def body(t, _):
      for b in range(NBUF):
        j = t * NBUF + b
        wait_gather(b, j)
        pltpu.async_copy(rows_v.at[b], acc.at[dst_v.at[j]], ssem, add=True)
        @pl.when(jnp.logical_and(j >= 1, j + 4 < n_chunks))
        def _():
          drain_scatter(b)
        @pl.when(j + 4 < n_chunks)
        def _():
          pltpu.async_copy(my_msg.at[src_v.at[j + 4]],
                           rows_v.at[(b + 4) % NBUF], gsem)
      return 0
    lax.fori_loop(0, n_chunks // NBUF, body, 0)
    for b in range(NBUF):
      drain_scatter(b)

    plsc.subcore_barrier()
    obase = sid * out_per
    pltpu.sync_copy(acc.at[pl.ds(obase, out_per)],
                    out_hbm.at[cid, pl.ds(obase, out_per)])
    if tail:
      @pl.when(sid == NS - 1)
      def _():
        pltpu.sync_copy(acc.at[pl.ds(out_per * NS, tail)],
                        out_hbm.at[cid, pl.ds(out_per * NS, tail)])

  return sc_scatter


def _mm(a, w):
  return lax.dot_general(a, w, (((1,), (0,)), ((), ())),
                         preferred_element_type=jnp.float32)


def _split_msg(m, m_ref, d2):
  m_ref[0] = m[:, :d2]
  m_ref[1] = m[:, d2:]


def _tc_init(x, w_in, b_in, w_msg0, b_msg0, block):
  n, d = x.shape
  d2 = d // 2

  def body(x_ref, wi_ref, bi_ref, wm_ref, bm_ref, s_ref, m_ref):
    s = jax.nn.relu(_mm(x_ref[...], wi_ref[...]) + bi_ref[...])
    s_ref[...] = s
    _split_msg(jax.nn.relu(_mm(s, wm_ref[...]) + bm_ref[...]), m_ref, d2)

  grid = n // block
  full = lambda shape: pl.BlockSpec(shape, lambda i: (0, 0))
  return pl.pallas_call(
      body,
      grid=(grid,),
      in_specs=[
          pl.BlockSpec((block, d), lambda i: (i, 0)),
          full((d, d)), full((1, d)), full((d, d)), full((1, d)),
      ],
      out_specs=[pl.BlockSpec((block, d), lambda i: (i, 0)),
                 pl.BlockSpec((NC, block, d2), lambda i: (0, i, 0))],
      out_shape=[jax.ShapeDtypeStruct((n, d), jnp.float32),
                 jax.ShapeDtypeStruct((NC, n, d2), jnp.float32)],
  )(x, w_in, b_in.reshape(1, d), w_msg0, b_msg0.reshape(1, d))


def _tc_gru(state, agg, w_sa, w_aa, b_ru, wt_s, wt_b, b_t, wm, bm, block):
  n, d = state.shape
  d2 = d // 2
  emit_msg = wm is not None

  def body(s_ref, a_ref, wsa_ref, waa_ref, bru_ref, wts_ref, wtb_ref,
           bt_ref, *rest):
    if emit_msg:
      wm_ref, bm_ref, ns_ref, m_ref = rest
    else:
      (ns_ref,) = rest
    s = s_ref[...]
    a = jnp.concatenate([a_ref[0], a_ref[1]], axis=1)
    ru = jax.nn.sigmoid(_mm(s, wsa_ref[...]) + _mm(a, waa_ref[...])
                        + bru_ref[...])
    r = ru[:, :d]
    u = ru[:, d:]
    c = jnp.tanh(_mm(s * r, wts_ref[...]) + _mm(a, wtb_ref[...]) + bt_ref[...])
    ns = s * (1.0 - u) + c * u
    ns_ref[...] = ns
    if emit_msg:
      _split_msg(jax.nn.relu(_mm(ns, wm_ref[...]) + bm_ref[...]), m_ref, d2)

  grid = n // block
  full = lambda shape: pl.BlockSpec(shape, lambda i: (0, 0))
  in_specs = [
      pl.BlockSpec((block, d), lambda i: (i, 0)),
      pl.BlockSpec((NC, block, d2), lambda i: (0, i, 0)),
      full((d, 2 * d)), full((d, 2 * d)), full((1, 2 * d)),
      full((d, d)), full((d, d)), full((1, d)),
  ]
  args = [state, agg, w_sa, w_aa, b_ru.reshape(1, 2 * d),
          wt_s, wt_b, b_t.reshape(1, d)]
  if emit_msg:
    in_specs += [full((d, d)), full((1, d))]
    args += [wm, bm.reshape(1, d)]
    out_specs = [pl.BlockSpec((block, d), lambda i: (i, 0)),
                 pl.BlockSpec((NC, block, d2), lambda i: (0, i, 0))]
    out_shape = [jax.ShapeDtypeStruct((n, d), jnp.float32),
                 jax.ShapeDtypeStruct((NC, n, d2), jnp.float32)]
  else:
    out_specs = [pl.BlockSpec((block, d), lambda i: (i, 0))]
    out_shape = [jax.ShapeDtypeStruct((n, d), jnp.float32)]
  return pl.pallas_call(
      body, grid=(grid,), in_specs=in_specs,
      out_specs=out_specs, out_shape=out_shape,
  )(*args)


def kernel(x, edge_index, batch, w_in, b_in, w_msg, b_msg,
           w_r, b_r, w_u, b_u, w_t, b_t):
  n, d = x.shape
  d2 = d // 2
  e = edge_index.shape[1]
  rounds = w_msg.shape[0]
  block = 1000 if n % 1000 == 0 else n // 8

  n_chunks = -(-e // (NS * CHUNK * NBUF)) * NBUF
  pe = NS * n_chunks * CHUNK
  pad = pe - e
  src = edge_index[0]
  dst = edge_index[1]
  if pad:
    fill = jnp.arange(pad, dtype=jnp.int32) % 16
    src = jnp.concatenate([src, fill])
    dst = jnp.concatenate([dst, n + fill])
  src_p = src.reshape(NS, n_chunks, CHUNK)
  dst_p = dst.reshape(NS, n_chunks, CHUNK)
  rows_pad = n + 16
  sc_scatter = _make_sc_scatter(n, d2, n_chunks, rows_pad)

  w_sa = jnp.concatenate([w_r[:, :d, :], w_u[:, :d, :]], axis=2)
  w_aa = jnp.concatenate([w_r[:, d:, :], w_u[:, d:, :]], axis=2)
  b_ru = jnp.concatenate([b_r, b_u], axis=1)
  wt_s = w_t[:, :d, :]
  wt_b = w_t[:, d:, :]

  state, msg = _tc_init(x, w_in, b_in, w_msg[0], b_msg[0], block)
  for r in range(rounds):
    agg = sc_scatter(msg, src_p, dst_p)
    last = r == rounds - 1
    out = _tc_gru(state, agg, w_sa[r], w_aa[r], b_ru[r], wt_s[r], wt_b[r],
                  b_t[r], None if last else w_msg[r + 1],
                  None if last else b_msg[r + 1], block)
    if last:
      state = out[0]
    else:
      state, msg = out
  return state


# speedup vs baseline: 1.3574x; 1.2525x over previous
import functools

import jax
import jax.numpy as jnp
from jax import lax
from jax.experimental import pallas as pl
from jax.experimental.pallas import tpu as pltpu
from jax.experimental.pallas import tpu_sc as plsc

NC = 2
NS = 16
CHUNK = 128
NBUF = 5


def _make_sc_scatter(n, d2, n_chunks, rows_pad):
  rows_per_tile = rows_pad // NS
  out_per = (n // NS) // 8 * 8
  tail = n - out_per * NS
  mesh = plsc.VectorSubcoreMesh(core_axis_name="c", subcore_axis_name="s")

  @functools.partial(
      pl.kernel,
      out_type=jax.ShapeDtypeStruct((2 * n, d2), jnp.float32),
      mesh=mesh,
      scratch_types=[
          pltpu.VMEM((n_chunks, CHUNK), jnp.int32),
          pltpu.VMEM((n_chunks, CHUNK), jnp.int32),
          pltpu.VMEM((NBUF, CHUNK, d2), jnp.float32),
          pltpu.VMEM_SHARED((rows_pad, d2), jnp.float32),
          pltpu.SemaphoreType.DMA,
          pltpu.SemaphoreType.DMA,
      ],
      compiler_params=pltpu.CompilerParams(use_tc_tiling_on_sc=False),
  )
  def sc_scatter(msg_hbm, src_hbm, dst_hbm, out_hbm,
                 src_v, dst_v, rows_v, acc, gsem, ssem):
    cid = lax.axis_index("c")
    sid = lax.axis_index("s")
    my_msg = msg_hbm

    idx_cp = (pltpu.async_copy(src_hbm.at[sid], src_v, gsem),
              pltpu.async_copy(dst_hbm.at[sid], dst_v, gsem))

    zeros_v = rows_v.at[0, pl.ds(0, 16)]
    def zrow(i, _):
      def zcol(j, _):
        rows_v[0, i, pl.ds(j * 16, 16)] = jnp.zeros((16,), jnp.float32)
        return 0
      return lax.fori_loop(0, d2 // 16, zcol, 0)
    lax.fori_loop(0, 16, zrow, 0)

    base = sid * rows_per_tile
    def zacc(t, _):
      pltpu.sync_copy(zeros_v, acc.at[pl.ds(base + t * 16, 16)])
      return 0
    lax.fori_loop(0, rows_per_tile // 16, zacc, 0)
    if rows_per_tile % 16:
      pltpu.sync_copy(zeros_v, acc.at[pl.ds(base + rows_per_tile - 16, 16)])
    for cp in idx_cp:
      cp.wait()

    def fix_chunk(i):
      for v in range(CHUNK // 16):
        sl = pl.ds(v * 16, 16)
        src_v[i, sl] = src_v[i, sl] * 2 + cid
    def fix_head(i, _):
      fix_chunk(i)
      return 0
    lax.fori_loop(0, 4, fix_head, 0)
    for b in range(4):
      pltpu.async_copy(my_msg.at[src_v.at[b]], rows_v.at[b], gsem)
    def fix_rest(i, _):
      fix_chunk(i)
      return 0
    lax.fori_loop(4, n_chunks, fix_rest, 0)
    plsc.subcore_barrier()


    def wait_gather(b, j):
      pltpu.make_async_copy(my_msg.at[src_v.at[j]], rows_v.at[b], gsem).wait()

    def drain_scatter(b):
      pltpu.make_async_copy(rows_v.at[b], acc.at[dst_v.at[0]], ssem).wait()

    def body(t, _):
      for b in range(NBUF):
        j = t * NBUF + b
        wait_gather(b, j)
        pltpu.async_copy(rows_v.at[b], acc.at[dst_v.at[j]], ssem, add=True)
        @pl.when(jnp.logical_and(j >= 1, j + 4 < n_chunks))
        def _():
          drain_scatter(b)
        @pl.when(j + 4 < n_chunks)
        def _():
          pltpu.async_copy(my_msg.at[src_v.at[j + 4]],
                           rows_v.at[(b + 4) % NBUF], gsem)
      return 0
    lax.fori_loop(0, n_chunks // NBUF, body, 0)
    for b in range(NBUF):
      drain_scatter(b)

    plsc.subcore_barrier()
    obase = sid * out_per
    nblk = -(-out_per // CHUNK)
    offs = [i * CHUNK for i in range(nblk - 1)] + [out_per - CHUNK]

    def emit_out(buf, row0):
      for v in range(CHUNK // 16):
        dst_v[buf, pl.ds(v * 16, 16)] = (
            (row0 + v * 16 + lax.iota(jnp.int32, 16)) * 2 + cid)
      pltpu.sync_copy(acc.at[pl.ds(row0, CHUNK)], rows_v.at[buf])
      return pltpu.async_copy(rows_v.at[buf], out_hbm.at[dst_v.at[buf]], ssem)

    cps = [emit_out(i % NBUF, obase + off) for i, off in enumerate(offs)]
    for cp in cps:
      cp.wait()
    if tail:
      @pl.when(sid == NS - 1)
      def _():
        emit_out(0, n - CHUNK).wait()

  return sc_scatter


def _mm(a, w):
  return lax.dot_general(a, w, (((1,), (0,)), ((), ())),
                         preferred_element_type=jnp.float32)


def _tc_init(x, w_in, b_in, w_msg0, b_msg0, block):
  n, d = x.shape

  def body(x_ref, wi_ref, bi_ref, wm_ref, bm_ref, s_ref, m_ref):
    s = jax.nn.relu(_mm(x_ref[...], wi_ref[...]) + bi_ref[...])
    s_ref[...] = s
    m_ref[...] = jax.nn.relu(_mm(s, wm_ref[...]) + bm_ref[...])

  grid = n // block
  full = lambda shape: pl.BlockSpec(shape, lambda i: (0, 0))
  return pl.pallas_call(
      body,
      grid=(grid,),
      in_specs=[
          pl.BlockSpec((block, d), lambda i: (i, 0)),
          full((d, d)), full((1, d)), full((d, d)), full((1, d)),
      ],
      out_specs=[pl.BlockSpec((block, d), lambda i: (i, 0))] * 2,
      out_shape=[jax.ShapeDtypeStruct((n, d), jnp.float32)] * 2,
  )(x, w_in, b_in.reshape(1, d), w_msg0, b_msg0.reshape(1, d))


def _tc_gru(state, agg, w_sa, w_aa, b_ru, wt_s, wt_b, b_t, wm, bm, block):
  n, d = state.shape
  d2 = d // 2
  emit_msg = wm is not None

  def body(s_ref, a_ref, wsa_ref, waa_ref, bru_ref, wts_ref, wtb_ref,
           bt_ref, *rest):
    if emit_msg:
      wm_ref, bm_ref, ns_ref, m_ref = rest
    else:
      (ns_ref,) = rest
    s = s_ref[...]
    a = a_ref[...]
    ru = jax.nn.sigmoid(_mm(s, wsa_ref[...]) + _mm(a, waa_ref[...])
                        + bru_ref[...])
    r = ru[:, :d]
    u = ru[:, d:]
    c = jnp.tanh(_mm(s * r, wts_ref[...]) + _mm(a, wtb_ref[...]) + bt_ref[...])
    ns = s * (1.0 - u) + c * u
    ns_ref[...] = ns
    if emit_msg:
      m_ref[...] = jax.nn.relu(_mm(ns, wm_ref[...]) + bm_ref[...])

  grid = n // block
  full = lambda shape: pl.BlockSpec(shape, lambda i: (0, 0))
  in_specs = [
      pl.BlockSpec((block, d), lambda i: (i, 0)),
      pl.BlockSpec((block, d), lambda i: (i, 0)),
      full((d, 2 * d)), full((d, 2 * d)), full((1, 2 * d)),
      full((d, d)), full((d, d)), full((1, d)),
  ]
  args = [state, agg, w_sa, w_aa, b_ru.reshape(1, 2 * d),
          wt_s, wt_b, b_t.reshape(1, d)]
  if emit_msg:
    in_specs += [full((d, d)), full((1, d))]
    args += [wm, bm.reshape(1, d)]
  n_out = 2 if emit_msg else 1
  out_specs = [pl.BlockSpec((block, d), lambda i: (i, 0))] * n_out
  out_shape = [jax.ShapeDtypeStruct((n, d), jnp.float32)] * n_out
  return pl.pallas_call(
      body, grid=(grid,), in_specs=in_specs,
      out_specs=out_specs, out_shape=out_shape,
  )(*args)


def kernel(x, edge_index, batch, w_in, b_in, w_msg, b_msg,
           w_r, b_r, w_u, b_u, w_t, b_t):
  n, d = x.shape
  d2 = d // 2
  e = edge_index.shape[1]
  rounds = w_msg.shape[0]
  block = 2000 if n % 2000 == 0 else n // 4

  n_chunks = -(-e // (NS * CHUNK * NBUF)) * NBUF
  pe = NS * n_chunks * CHUNK
  pad = pe - e
  src = edge_index[0]
  dst = edge_index[1]
  if pad:
    fill = jnp.arange(pad, dtype=jnp.int32) % 16
    src = jnp.concatenate([src, fill])
    dst = jnp.concatenate([dst, n + fill])
  src_p = src.reshape(NS, n_chunks, CHUNK)
  dst_p = dst.reshape(NS, n_chunks, CHUNK)
  rows_pad = n + 16
  sc_scatter = _make_sc_scatter(n, d2, n_chunks, rows_pad)

  w_sa = jnp.concatenate([w_r[:, :d, :], w_u[:, :d, :]], axis=2)
  w_aa = jnp.concatenate([w_r[:, d:, :], w_u[:, d:, :]], axis=2)
  b_ru = jnp.concatenate([b_r, b_u], axis=1)
  wt_s = w_t[:, :d, :]
  wt_b = w_t[:, d:, :]

  state, msg = _tc_init(x, w_in, b_in, w_msg[0], b_msg[0], block)
  for r in range(rounds):
    agg = sc_scatter(msg.reshape(2 * n, d2), src_p, dst_p).reshape(n, d)
    last = r == rounds - 1
    out = _tc_gru(state, agg, w_sa[r], w_aa[r], b_ru[r], wt_s[r], wt_b[r],
                  b_t[r], None if last else w_msg[r + 1],
                  None if last else b_msg[r + 1], block)
    if last:
      state = out[0]
    else:
      state, msg = out
  return state
